# baseline (speedup 1.0000x reference)
import functools

import jax
import jax.numpy as jnp
from jax import lax
from jax.experimental import pallas as pl
from jax.experimental.pallas import tpu as pltpu
from jax.experimental.pallas import tpu_sc as plsc

N = 25000
D = 64
NE = 800000
QR = 5
BS = 1024
TEMP = 0.2
LAM1 = 0.2
LAM2 = 1e-07

NSC = 2
NTILE = 16
CH = 64
CPT = 784
GS = 28
NG = CPT // GS
RD = 4
NEP = NTILE * CPT * CH
DUMMY = N
NT = 25088
RPT = NT // NTILE

_f32 = jnp.float32


def _mesh():
    return plsc.VectorSubcoreMesh(
        core_axis_name="c", subcore_axis_name="s", num_cores=NSC,
        num_subcores=NTILE)


@functools.cache
def _build_sc_degrees():
    return functools.partial(
        pl.kernel,
        out_type=[jax.ShapeDtypeStruct((NT,), _f32),
                  jax.ShapeDtypeStruct((NT,), _f32)],
        mesh=_mesh(),
        scratch_types=[
            pltpu.VMEM((CPT, CH), jnp.int32),
            pltpu.VMEM((CH,), _f32),
            pltpu.VMEM((1024,), _f32),
            pltpu.VMEM_SHARED((NT,), _f32),
            pltpu.SemaphoreType.DMA,
            pltpu.SemaphoreType.DMA,
            pltpu.SemaphoreType.DMA,
        ],
    )(_sc_degrees_body)


def _sc_degrees_body(eu, ei, deg_u, deg_i, idx_all, ones_v, zbuf_v, acc_sh,
                     ssem_a, ssem_b, lsem):
    cid = lax.axis_index("c")
    wid = lax.axis_index("s")

    def run(src, out):
        dl = pltpu.async_copy(src.at[pl.ds(wid * CPT, CPT)], idx_all, lsem)

        def fill1(j, c):
            ones_v[pl.ds(j * 16, 16)] = jnp.full((16,), 1.0, _f32)
            return c
        lax.fori_loop(0, CH // 16, fill1, 0)

        def fill0(j, c):
            zbuf_v[pl.ds(j * 16, 16)] = jnp.zeros((16,), _f32)
            return c
        lax.fori_loop(0, 1024 // 16, fill0, 0)

        base = wid * RPT
        pltpu.sync_copy(zbuf_v, acc_sh.at[pl.ds(base, 1024)])
        pltpu.sync_copy(zbuf_v.at[pl.ds(0, 512)],
                        acc_sh.at[pl.ds(base + 1024, 512)])
        pltpu.sync_copy(zbuf_v.at[pl.ds(0, 32)],
                        acc_sh.at[pl.ds(base + 1536, 32)])
        dl.wait()
        plsc.subcore_barrier()

        def body(v, c):
            ca = 2 * v
            cb = ca + 1

            @pl.when(v > 0)
            def _():
                pltpu.make_async_copy(
                    ones_v, acc_sh.at[idx_all.at[ca]], ssem_a).wait()
                pltpu.make_async_copy(
                    ones_v, acc_sh.at[idx_all.at[cb]], ssem_b).wait()

            pltpu.async_copy(ones_v, acc_sh.at[idx_all.at[ca]], ssem_a,
                             add=True)
            pltpu.async_copy(ones_v, acc_sh.at[idx_all.at[cb]], ssem_b,
                             add=True)
            return c
        lax.fori_loop(0, CPT // 2, body, 0)
        pltpu.make_async_copy(ones_v, acc_sh.at[idx_all.at[0]], ssem_a).wait()
        pltpu.make_async_copy(ones_v, acc_sh.at[idx_all.at[0]], ssem_b).wait()
        plsc.subcore_barrier()
        pltpu.sync_copy(acc_sh.at[pl.ds(base, 1024)], zbuf_v)
        pltpu.sync_copy(zbuf_v, out.at[pl.ds(base, 1024)])
        pltpu.sync_copy(acc_sh.at[pl.ds(base + 1024, 512)],
                        zbuf_v.at[pl.ds(0, 512)])
        pltpu.sync_copy(zbuf_v.at[pl.ds(0, 512)],
                        out.at[pl.ds(base + 1024, 512)])
        pltpu.sync_copy(acc_sh.at[pl.ds(base + 1536, 32)],
                        zbuf_v.at[pl.ds(0, 32)])
        pltpu.sync_copy(zbuf_v.at[pl.ds(0, 32)],
                        out.at[pl.ds(base + 1536, 32)])

    @pl.when(cid == 0)
    def _():
        run(eu, deg_u)

    @pl.when(cid == 1)
    def _():
        run(ei, deg_i)


@functools.cache
def _build_sc_spmm():
    return functools.partial(
        pl.kernel,
        out_type=[jax.ShapeDtypeStruct((NT, D), _f32),
                  jax.ShapeDtypeStruct((NT, D), _f32)],
        mesh=_mesh(),
        scratch_types=[
            pltpu.VMEM((2, GS, CH), jnp.int32),
            pltpu.VMEM((2, GS, CH), jnp.int32),
            pltpu.VMEM((RD, CH, D), _f32),
            pltpu.VMEM_SHARED((NT, D), _f32),
        ] + [pltpu.SemaphoreType.DMA] * (4 + 2 * RD),
        compiler_params=pltpu.CompilerParams(use_tc_tiling_on_sc=False),
    )(_sc_spmm_body)


def _sc_spmm_body(t_u, t_i, eu, ei, r_u, r_i, idxs_v, idxd_v, rows_v,
                  acc_sh, *sems):
    cid = lax.axis_index("c")
    wid = lax.axis_index("s")
    isem = sems[0:2]
    jsem = sems[2:4]
    gsem = sems[4:4 + RD]
    ssem = sems[4 + RD:4 + 2 * RD]

    def run(table, src, dst, out):
        def load_idx(g, s):
            row0 = wid * CPT + g * GS
            pltpu.async_copy(src.at[pl.ds(row0, GS)], idxs_v.at[s], isem[s])
            pltpu.async_copy(dst.at[pl.ds(row0, GS)], idxd_v.at[s], jsem[s])

        def wait_idx(s):
            pltpu.make_async_copy(src.at[pl.ds(0, GS)], idxs_v.at[s],
                                  isem[s]).wait()
            pltpu.make_async_copy(dst.at[pl.ds(0, GS)], idxd_v.at[s],
                                  jsem[s]).wait()

        def process_group(s, guard, mid_cb):
            descs = [None] * GS
            for j in range(GS):
                rb = j % RD
                if j >= RD:
                    pltpu.make_async_copy(
                        rows_v.at[rb], acc_sh.at[idxd_v.at[s, j - RD]],
                        ssem[rb]).wait()
                else:
                    def _w(s=s, j=j, rb=rb):
                        pltpu.make_async_copy(
                            rows_v.at[rb],
                            acc_sh.at[idxd_v.at[1 - s, GS - RD + j]],
                            ssem[rb]).wait()
                    if guard is None:
                        _w()
                    else:
                        pl.when(guard)(_w)
                descs[j] = pltpu.async_copy(
                    table.at[idxs_v.at[s, j]], rows_v.at[rb], gsem[rb])
                if j == RD and mid_cb is not None:
                    mid_cb()
                if j >= 2:
                    pb = (j - 2) % RD
                    descs[j - 2].wait()
                    pltpu.async_copy(rows_v.at[pb],
                                     acc_sh.at[idxd_v.at[s, j - 2]],
                                     ssem[pb], add=True)
            for jj in (GS - 2, GS - 1):
                descs[jj].wait()
                pltpu.async_copy(rows_v.at[jj % RD],
                                 acc_sh.at[idxd_v.at[s, jj]],
                                 ssem[jj % RD], add=True)

        load_idx(0, 0)

        def fill0(r, c):
            for q in range(D // 16):
                rows_v[0, r, pl.ds(q * 16, 16)] = jnp.zeros((16,), _f32)
            return c
        lax.fori_loop(0, CH, fill0, 0)
        base = wid * RPT
        for t in range(RPT // CH):
            pltpu.sync_copy(rows_v.at[0], acc_sh.at[pl.ds(base + t * CH, CH)])
        pltpu.sync_copy(rows_v.at[0, pl.ds(0, RPT % CH)],
                        acc_sh.at[pl.ds(base + (RPT // CH) * CH, RPT % CH)])
        plsc.subcore_barrier()

        wait_idx(0)

        def outer(t, c):
            ga = 2 * t
            process_group(0, guard=t > 0,
                          mid_cb=lambda: load_idx(ga + 1, 1))
            wait_idx(1)

            def mid1():
                @pl.when(ga + 2 < NG)
                def _():
                    load_idx(ga + 2, 0)
            process_group(1, guard=None, mid_cb=mid1)

            @pl.when(ga + 2 < NG)
            def _():
                wait_idx(0)
            return c
        lax.fori_loop(0, NG // 2, outer, 0)
        for j in range(GS - RD, GS):
            pltpu.make_async_copy(rows_v.at[j % RD],
                                  acc_sh.at[idxd_v.at[1, j]],
                                  ssem[j % RD]).wait()
        plsc.subcore_barrier()
        for t in range(RPT // CH):
            pltpu.sync_copy(acc_sh.at[pl.ds(base + t * CH, CH)], rows_v.at[0])
            pltpu.sync_copy(rows_v.at[0], out.at[pl.ds(base + t * CH, CH)])
        tail = base + (RPT // CH) * CH
        pltpu.sync_copy(acc_sh.at[pl.ds(tail, RPT % CH)],
                        rows_v.at[0, pl.ds(0, RPT % CH)])
        pltpu.sync_copy(rows_v.at[0, pl.ds(0, RPT % CH)],
                        out.at[pl.ds(tail, RPT % CH)])

    @pl.when(cid == 0)
    def _():
        run(t_i, ei, eu, r_u)

    @pl.when(cid == 1)
    def _():
        run(t_u, eu, ei, r_i)


_BLKA = 3136
_GRIDA = NT // _BLKA


def _prep_body(degu, degi, e0u, e0i, tu1, ti1, a_o, b_o, invu_o, invi_o):
    du = jnp.maximum(degu[...], 1.0)
    di = jnp.maximum(degi[...], 1.0)
    av = lax.rsqrt(du)
    bv = lax.rsqrt(di)
    a_o[...] = av
    b_o[...] = bv
    invu_o[...] = 1.0 / du
    invi_o[...] = 1.0 / di
    tu1[...] = av * e0u[...]
    ti1[...] = bv * e0i[...]


def _tc_prep(deg_u, deg_i, e0u_p, e0i_p):
    vec = pl.BlockSpec((_BLKA, 1), lambda i: (i, 0))
    mat = pl.BlockSpec((_BLKA, D), lambda i: (i, 0))
    return pl.pallas_call(
        _prep_body,
        grid=(_GRIDA,),
        in_specs=[vec, vec, mat, mat],
        out_specs=[mat, mat, vec, vec, vec, vec],
        out_shape=[jax.ShapeDtypeStruct((NT, D), _f32),
                   jax.ShapeDtypeStruct((NT, D), _f32)] +
                  [jax.ShapeDtypeStruct((NT, 1), _f32)] * 4,
    )(deg_u.reshape(NT, 1), deg_i.reshape(NT, 1), e0u_p, e0i_p)


def _scale2_body(ru1, ri1, invu, invi, tu2, ti2):
    tu2[...] = invu[...] * ru1[...]
    ti2[...] = invi[...] * ri1[...]


def _tc_scale2(r_u1, r_i1, inv_u, inv_i):
    vec = pl.BlockSpec((_BLKA, 1), lambda i: (i, 0))
    mat = pl.BlockSpec((_BLKA, D), lambda i: (i, 0))
    return pl.pallas_call(
        _scale2_body,
        grid=(_GRIDA,),
        in_specs=[mat, mat, vec, vec],
        out_specs=[mat, mat],
        out_shape=[jax.ShapeDtypeStruct((NT, D), _f32)] * 2,
    )(r_u1, r_i1, inv_u, inv_i)


QP = 8
_BLKC = 1568
_GRIDC = NT // _BLKC


def _c_body(e0u, ru1, ru2, a_r, utT, e0i, ri1, ri2, b_r, vtT,
            e0u_g, a_g, ru1_g, ru2_g, ums_g, vms_g, e0i3, b3, ri13, ri23,
            loss_o, lossr_o, losss_o,
            esu_s, esi_s, wu_s, wi_s, reg_s, se_u, se_i):
    i = pl.program_id(0)

    @pl.when(i < _GRIDC)
    def _():
        av = a_r[...]
        bv = b_r[...]
        e0u_v = e0u[...]
        e0i_v = e0i[...]
        ru1_v = ru1[...]
        ri1_v = ri1[...]
        esu_s[pl.ds(i * _BLKC, _BLKC), :] = e0u_v + av * (ru1_v + ru2[...])
        esi_s[pl.ds(i * _BLKC, _BLKC), :] = e0i_v + bv * (ri1_v + ri2[...])
        xu = e0u_v + av * ru1_v
        xi = e0i_v + bv * ri1_v
        dn = (((0,), (0,)), ((), ()))
        wu_p = lax.dot_general(utT[...], xu, dn, preferred_element_type=_f32)
        wi_p = lax.dot_general(vtT[...], xi, dn, preferred_element_type=_f32)
        reg_p = jnp.sum(e0u_v * e0u_v) + jnp.sum(e0i_v * e0i_v)

        @pl.when(i == 0)
        def _():
            wu_s[...] = jnp.zeros_like(wu_s)
            wi_s[...] = jnp.zeros_like(wi_s)
            reg_s[...] = jnp.zeros_like(reg_s)

        wu_s[...] += wu_p
        wi_s[...] += wi_p
        reg_s[...] += jnp.reshape(reg_p, (1, 1))

    @pl.when(i >= _GRIDC)
    def _():
        j = i - _GRIDC
        blku = esu_s[pl.ds(j * _BLKC, _BLKC), :]
        blki = esi_s[pl.ds(j * _BLKC, _BLKC), :]
        gu = e0u_g[...] + jnp.dot(ums_g[...], wi_s[...],
                                  preferred_element_type=_f32)
        gi = e0i3[0:BS, :] + jnp.dot(vms_g[...], wu_s[...],
                                     preferred_element_type=_f32)
        dn_nn = (((1,), (1,)), ((), ()))
        lg_u = lax.dot_general(gu, blku, dn_nn,
                               preferred_element_type=_f32) / TEMP
        lg_i = lax.dot_general(gi, blki, dn_nn,
                               preferred_element_type=_f32) / TEMP
        seu_p = jnp.sum(jnp.exp(lg_u), axis=1, keepdims=True)
        sei_p = jnp.sum(jnp.exp(lg_i), axis=1, keepdims=True)

        @pl.when(i == _GRIDC)
        def _():
            se_u[...] = jnp.zeros_like(se_u)
            se_i[...] = jnp.zeros_like(se_i)

        se_u[...] += seu_p
        se_i[...] += sei_p

        @pl.when(i == 2 * _GRIDC - 1)
        def _():
            eusu = e0u_g[...] + a_g[...] * (ru1_g[...] + ru2_g[...])
            ei3 = e0i3[...] + b3[...] * (ri13[...] + ri23[...])
            eisi = ei3[0:BS, :]
            eisp = ei3[BS:2 * BS, :]
            eisn = ei3[2 * BS:3 * BS, :]
            pad = float(NT - N)
            neg = jnp.mean(jnp.log(se_u[...] - pad + 1e-08))
            neg += jnp.mean(jnp.log(se_i[...] - pad + 1e-08))
            pos = jnp.mean(jnp.clip(jnp.sum(gu * eusu, axis=1) / TEMP,
                                    -5.0, 5.0))
            pos += jnp.mean(jnp.clip(jnp.sum(gi * eisi, axis=1) / TEMP,
                                     -5.0, 5.0))
            loss_s = neg - pos
            ps = jnp.sum(eusu * eisp, axis=1)
            ns = jnp.sum(eusu * eisn, axis=1)
            sig = 1.0 / (1.0 + jnp.exp(-(ps - ns)))
            loss_r = -jnp.mean(jnp.log(sig + 1e-08))
            lossr_o[...] = jnp.reshape(loss_r, (1, 1))
            losss_o[...] = jnp.reshape(LAM1 * loss_s, (1, 1))
            loss_o[...] = jnp.reshape(loss_r + LAM1 * loss_s, (1, 1)) \
                + LAM2 * reg_s[...]


def _tc_c(e0u_p, r_u1, r_u2, a_v, utT_p, e0i_p, r_i1, r_i2, b_v, vtT_p,
          e0u_g, a_g, ru1_g, ru2_g, ums_g, vms_g, e0i3, b3, ri13, ri23):
    vec = pl.BlockSpec((_BLKC, 1), lambda i: (i % _GRIDC, 0))
    mat = pl.BlockSpec((_BLKC, D), lambda i: (i % _GRIDC, 0))
    fac = pl.BlockSpec((_BLKC, QP), lambda i: (i % _GRIDC, 0))
    cst = lambda shape: pl.BlockSpec(shape, lambda i: (0, 0))
    return pl.pallas_call(
        _c_body,
        grid=(2 * _GRIDC,),
        in_specs=[mat, mat, mat, vec, fac, mat, mat, mat, vec, fac,
                  cst((BS, D)), cst((BS, 1)), cst((BS, D)), cst((BS, D)),
                  cst((BS, QP)), cst((BS, QP)),
                  cst((3 * BS, D)), cst((3 * BS, 1)),
                  cst((3 * BS, D)), cst((3 * BS, D))],
        out_specs=[cst((1, 1)), cst((1, 1)), cst((1, 1))],
        out_shape=[jax.ShapeDtypeStruct((1, 1), _f32)] * 3,
        scratch_shapes=[pltpu.VMEM((NT, D), _f32),
                        pltpu.VMEM((NT, D), _f32),
                        pltpu.VMEM((QP, D), _f32),
                        pltpu.VMEM((QP, D), _f32),
                        pltpu.VMEM((1, 1), _f32),
                        pltpu.VMEM((BS, 1), _f32),
                        pltpu.VMEM((BS, 1), _f32)],
    )(e0u_p, r_u1, r_u2, a_v, utT_p, e0i_p, r_i1, r_i2, b_v, vtT_p,
      e0u_g, a_g, ru1_g, ru2_g, ums_g, vms_g, e0i3, b3, ri13, ri23)


def kernel(uids, iids, pos, neg, edge_u, edge_i, edge_vals,
           E_u_0, E_i_0, u_mul_s, v_mul_s, ut, vt):
    del edge_vals
    padi = jnp.full((NEP - NE,), DUMMY, jnp.int32)
    eu_p = jnp.concatenate([edge_u.astype(jnp.int32), padi]) \
        .reshape(NEP // CH, CH)
    ei_p = jnp.concatenate([edge_i.astype(jnp.int32), padi]) \
        .reshape(NEP // CH, CH)
    e0u_p = jnp.pad(E_u_0, ((0, NT - N), (0, 0)))
    e0i_p = jnp.pad(E_i_0, ((0, NT - N), (0, 0)))

    deg_u, deg_i = _build_sc_degrees()(eu_p, ei_p)
    t_u1, t_i1, a_v, b_v, inv_u, inv_i = _tc_prep(deg_u, deg_i, e0u_p, e0i_p)
    sc_spmm = _build_sc_spmm()
    r_u1, r_i1 = sc_spmm(t_u1, t_i1, eu_p, ei_p)
    t_u2, t_i2 = _tc_scale2(r_u1, r_i1, inv_u, inv_i)
    r_u2, r_i2 = sc_spmm(t_u2, t_i2, eu_p, ei_p)

    utT_p = jnp.pad(ut.T, ((0, NT - N), (0, QP - QR)))
    vtT_p = jnp.pad(vt.T, ((0, NT - N), (0, QP - QR)))
    idx3 = jnp.concatenate([iids, pos, neg])
    loss, loss_r, loss_s = _tc_c(
        e0u_p, r_u1, r_u2, a_v, utT_p, e0i_p, r_i1, r_i2, b_v, vtT_p,
        E_u_0[uids], a_v[uids], r_u1[uids], r_u2[uids],
        jnp.pad(u_mul_s[uids], ((0, 0), (0, QP - QR))),
        jnp.pad(v_mul_s[iids], ((0, 0), (0, QP - QR))),
        E_i_0[idx3], b_v[idx3], r_i1[idx3], r_i2[idx3])
    return (loss[0, 0], loss_r[0, 0], loss_s[0, 0])


# speedup vs baseline: 1.0531x; 1.0531x over previous
import functools

import jax
import jax.numpy as jnp
from jax import lax
from jax.experimental import pallas as pl
from jax.experimental.pallas import tpu as pltpu
from jax.experimental.pallas import tpu_sc as plsc

N = 25000
D = 64
NE = 800000
QR = 5
BS = 1024
TEMP = 0.2
LAM1 = 0.2
LAM2 = 1e-07

NSC = 2
NTILE = 16
CH = 128
CPT = 392
GS = 14
NG = CPT // GS
RD = 2
LAG = 1
NEP = NTILE * CPT * CH
DUMMY = N
NT = 25088
RPT = NT // NTILE

_f32 = jnp.float32


def _mesh():
    return plsc.VectorSubcoreMesh(
        core_axis_name="c", subcore_axis_name="s", num_cores=NSC,
        num_subcores=NTILE)


@functools.cache
def _build_sc_degrees():
    return functools.partial(
        pl.kernel,
        out_type=[jax.ShapeDtypeStruct((NT,), _f32),
                  jax.ShapeDtypeStruct((NT,), _f32)],
        mesh=_mesh(),
        scratch_types=[
            pltpu.VMEM((CPT, CH), jnp.int32),
            pltpu.VMEM((CH,), _f32),
            pltpu.VMEM((1024,), _f32),
            pltpu.VMEM_SHARED((NT,), _f32),
            pltpu.SemaphoreType.DMA,
            pltpu.SemaphoreType.DMA,
            pltpu.SemaphoreType.DMA,
        ],
    )(_sc_degrees_body)


def _sc_degrees_body(eu, ei, deg_u, deg_i, idx_all, ones_v, zbuf_v, acc_sh,
                     ssem_a, ssem_b, lsem):
    cid = lax.axis_index("c")
    wid = lax.axis_index("s")

    def run(src, out):
        dl = pltpu.async_copy(src.at[pl.ds(wid * CPT, CPT)], idx_all, lsem)

        def fill1(j, c):
            ones_v[pl.ds(j * 16, 16)] = jnp.full((16,), 1.0, _f32)
            return c
        lax.fori_loop(0, CH // 16, fill1, 0)

        def fill0(j, c):
            zbuf_v[pl.ds(j * 16, 16)] = jnp.zeros((16,), _f32)
            return c
        lax.fori_loop(0, 1024 // 16, fill0, 0)

        base = wid * RPT
        pltpu.sync_copy(zbuf_v, acc_sh.at[pl.ds(base, 1024)])
        pltpu.sync_copy(zbuf_v.at[pl.ds(0, 512)],
                        acc_sh.at[pl.ds(base + 1024, 512)])
        pltpu.sync_copy(zbuf_v.at[pl.ds(0, 32)],
                        acc_sh.at[pl.ds(base + 1536, 32)])
        dl.wait()
        plsc.subcore_barrier()

        def body(v, c):
            ca = 2 * v
            cb = ca + 1

            @pl.when(v > 0)
            def _():
                pltpu.make_async_copy(
                    ones_v, acc_sh.at[idx_all.at[ca]], ssem_a).wait()
                pltpu.make_async_copy(
                    ones_v, acc_sh.at[idx_all.at[cb]], ssem_b).wait()

            pltpu.async_copy(ones_v, acc_sh.at[idx_all.at[ca]], ssem_a,
                             add=True)
            pltpu.async_copy(ones_v, acc_sh.at[idx_all.at[cb]], ssem_b,
                             add=True)
            return c
        lax.fori_loop(0, CPT // 2, body, 0)
        pltpu.make_async_copy(ones_v, acc_sh.at[idx_all.at[0]], ssem_a).wait()
        pltpu.make_async_copy(ones_v, acc_sh.at[idx_all.at[0]], ssem_b).wait()
        plsc.subcore_barrier()
        pltpu.sync_copy(acc_sh.at[pl.ds(base, 1024)], zbuf_v)
        pltpu.sync_copy(zbuf_v, out.at[pl.ds(base, 1024)])
        pltpu.sync_copy(acc_sh.at[pl.ds(base + 1024, 512)],
                        zbuf_v.at[pl.ds(0, 512)])
        pltpu.sync_copy(zbuf_v.at[pl.ds(0, 512)],
                        out.at[pl.ds(base + 1024, 512)])
        pltpu.sync_copy(acc_sh.at[pl.ds(base + 1536, 32)],
                        zbuf_v.at[pl.ds(0, 32)])
        pltpu.sync_copy(zbuf_v.at[pl.ds(0, 32)],
                        out.at[pl.ds(base + 1536, 32)])

    @pl.when(cid == 0)
    def _():
        run(eu, deg_u)

    @pl.when(cid == 1)
    def _():
        run(ei, deg_i)


@functools.cache
def _build_sc_spmm():
    return functools.partial(
        pl.kernel,
        out_type=[jax.ShapeDtypeStruct((NT, D), _f32),
                  jax.ShapeDtypeStruct((NT, D), _f32)],
        mesh=_mesh(),
        scratch_types=[
            pltpu.VMEM((2, GS, CH), jnp.int32),
            pltpu.VMEM((2, GS, CH), jnp.int32),
            pltpu.VMEM((RD, CH, D), _f32),
            pltpu.VMEM_SHARED((NT, D), _f32),
        ] + [pltpu.SemaphoreType.DMA] * (4 + 2 * RD),
        compiler_params=pltpu.CompilerParams(use_tc_tiling_on_sc=False),
    )(_sc_spmm_body)


def _sc_spmm_body(t_u, t_i, eu, ei, r_u, r_i, idxs_v, idxd_v, rows_v,
                  acc_sh, *sems):
    cid = lax.axis_index("c")
    wid = lax.axis_index("s")
    isem = sems[0:2]
    jsem = sems[2:4]
    gsem = sems[4:4 + RD]
    ssem = sems[4 + RD:4 + 2 * RD]

    def run(table, src, dst, out):
        def load_idx(g, s):
            row0 = wid * CPT + g * GS
            pltpu.async_copy(src.at[pl.ds(row0, GS)], idxs_v.at[s], isem[s])
            pltpu.async_copy(dst.at[pl.ds(row0, GS)], idxd_v.at[s], jsem[s])

        def wait_idx(s):
            pltpu.make_async_copy(src.at[pl.ds(0, GS)], idxs_v.at[s],
                                  isem[s]).wait()
            pltpu.make_async_copy(dst.at[pl.ds(0, GS)], idxd_v.at[s],
                                  jsem[s]).wait()

        def process_group(s, guard, mid_cb):
            descs = [None] * GS
            for j in range(GS):
                rb = j % RD
                if j >= RD:
                    pltpu.make_async_copy(
                        rows_v.at[rb], acc_sh.at[idxd_v.at[s, j - RD]],
                        ssem[rb]).wait()
                else:
                    def _w(s=s, j=j, rb=rb):
                        pltpu.make_async_copy(
                            rows_v.at[rb],
                            acc_sh.at[idxd_v.at[1 - s, GS - RD + j]],
                            ssem[rb]).wait()
                    if guard is None:
                        _w()
                    else:
                        pl.when(guard)(_w)
                descs[j] = pltpu.async_copy(
                    table.at[idxs_v.at[s, j]], rows_v.at[rb], gsem[rb])
                if j == RD and mid_cb is not None:
                    mid_cb()
                if j >= LAG:
                    pb = (j - LAG) % RD
                    descs[j - LAG].wait()
                    pltpu.async_copy(rows_v.at[pb],
                                     acc_sh.at[idxd_v.at[s, j - LAG]],
                                     ssem[pb], add=True)
            for jj in range(GS - LAG, GS):
                descs[jj].wait()
                pltpu.async_copy(rows_v.at[jj % RD],
                                 acc_sh.at[idxd_v.at[s, jj]],
                                 ssem[jj % RD], add=True)

        load_idx(0, 0)

        def fill0(r, c):
            for q in range(D // 16):
                rows_v[0, r, pl.ds(q * 16, 16)] = jnp.zeros((16,), _f32)
            return c
        lax.fori_loop(0, CH, fill0, 0)
        base = wid * RPT
        for t in range(RPT // CH):
            pltpu.sync_copy(rows_v.at[0], acc_sh.at[pl.ds(base + t * CH, CH)])
        pltpu.sync_copy(rows_v.at[0, pl.ds(0, RPT % CH)],
                        acc_sh.at[pl.ds(base + (RPT // CH) * CH, RPT % CH)])
        plsc.subcore_barrier()

        wait_idx(0)

        def outer(t, c):
            ga = 2 * t
            process_group(0, guard=t > 0,
                          mid_cb=lambda: load_idx(ga + 1, 1))
            wait_idx(1)

            def mid1():
                @pl.when(ga + 2 < NG)
                def _():
                    load_idx(ga + 2, 0)
            process_group(1, guard=None, mid_cb=mid1)

            @pl.when(ga + 2 < NG)
            def _():
                wait_idx(0)
            return c
        lax.fori_loop(0, NG // 2, outer, 0)
        for j in range(GS - RD, GS):
            pltpu.make_async_copy(rows_v.at[j % RD],
                                  acc_sh.at[idxd_v.at[1, j]],
                                  ssem[j % RD]).wait()
        plsc.subcore_barrier()
        for t in range(RPT // CH):
            pltpu.sync_copy(acc_sh.at[pl.ds(base + t * CH, CH)], rows_v.at[0])
            pltpu.sync_copy(rows_v.at[0], out.at[pl.ds(base + t * CH, CH)])
        tail = base + (RPT // CH) * CH
        pltpu.sync_copy(acc_sh.at[pl.ds(tail, RPT % CH)],
                        rows_v.at[0, pl.ds(0, RPT % CH)])
        pltpu.sync_copy(rows_v.at[0, pl.ds(0, RPT % CH)],
                        out.at[pl.ds(tail, RPT % CH)])

    @pl.when(cid == 0)
    def _():
        run(t_i, ei, eu, r_u)

    @pl.when(cid == 1)
    def _():
        run(t_u, eu, ei, r_i)


_BLKA = 3136
_GRIDA = NT // _BLKA


def _prep_body(degu, degi, e0u, e0i, tu1, ti1, a_o, b_o, invu_o, invi_o):
    du = jnp.maximum(degu[...], 1.0)
    di = jnp.maximum(degi[...], 1.0)
    av = lax.rsqrt(du)
    bv = lax.rsqrt(di)
    a_o[...] = av
    b_o[...] = bv
    invu_o[...] = 1.0 / du
    invi_o[...] = 1.0 / di
    tu1[...] = av * e0u[...]
    ti1[...] = bv * e0i[...]


def _tc_prep(deg_u, deg_i, e0u_p, e0i_p):
    vec = pl.BlockSpec((_BLKA, 1), lambda i: (i, 0))
    mat = pl.BlockSpec((_BLKA, D), lambda i: (i, 0))
    return pl.pallas_call(
        _prep_body,
        grid=(_GRIDA,),
        in_specs=[vec, vec, mat, mat],
        out_specs=[mat, mat, vec, vec, vec, vec],
        out_shape=[jax.ShapeDtypeStruct((NT, D), _f32),
                   jax.ShapeDtypeStruct((NT, D), _f32)] +
                  [jax.ShapeDtypeStruct((NT, 1), _f32)] * 4,
    )(deg_u.reshape(NT, 1), deg_i.reshape(NT, 1), e0u_p, e0i_p)


def _scale2_body(ru1, ri1, invu, invi, tu2, ti2):
    tu2[...] = invu[...] * ru1[...]
    ti2[...] = invi[...] * ri1[...]


def _tc_scale2(r_u1, r_i1, inv_u, inv_i):
    vec = pl.BlockSpec((_BLKA, 1), lambda i: (i, 0))
    mat = pl.BlockSpec((_BLKA, D), lambda i: (i, 0))
    return pl.pallas_call(
        _scale2_body,
        grid=(_GRIDA,),
        in_specs=[mat, mat, vec, vec],
        out_specs=[mat, mat],
        out_shape=[jax.ShapeDtypeStruct((NT, D), _f32)] * 2,
    )(r_u1, r_i1, inv_u, inv_i)


QP = 8
_BLKC = 1792
_GRIDC = NT // _BLKC


def _c_body(e0u, ru1, ru2, a_r, utT, e0i, ri1, ri2, b_r, vtT,
            e0u_g, a_g, ru1_g, ru2_g, ums_g, vms_g, e0i3, b3, ri13, ri23,
            loss_o, lossr_o, losss_o,
            esu_s, esi_s, wu_s, wi_s, reg_s, se_u, se_i):
    i = pl.program_id(0)

    @pl.when(i < _GRIDC)
    def _():
        av = a_r[...]
        bv = b_r[...]
        e0u_v = e0u[...]
        e0i_v = e0i[...]
        ru1_v = ru1[...]
        ri1_v = ri1[...]
        esu_s[pl.ds(i * _BLKC, _BLKC), :] = e0u_v + av * (ru1_v + ru2[...])
        esi_s[pl.ds(i * _BLKC, _BLKC), :] = e0i_v + bv * (ri1_v + ri2[...])
        xu = e0u_v + av * ru1_v
        xi = e0i_v + bv * ri1_v
        dn = (((1,), (0,)), ((), ()))
        wu_p = lax.dot_general(utT[...], xu, dn, preferred_element_type=_f32)
        wi_p = lax.dot_general(vtT[...], xi, dn, preferred_element_type=_f32)
        reg_p = jnp.sum(e0u_v * e0u_v) + jnp.sum(e0i_v * e0i_v)

        @pl.when(i == 0)
        def _():
            wu_s[...] = jnp.zeros_like(wu_s)
            wi_s[...] = jnp.zeros_like(wi_s)
            reg_s[...] = jnp.zeros_like(reg_s)

        wu_s[...] += wu_p
        wi_s[...] += wi_p
        reg_s[...] += jnp.reshape(reg_p, (1, 1))

    @pl.when(i >= _GRIDC)
    def _():
        j = i - _GRIDC
        blku = esu_s[pl.ds(j * _BLKC, _BLKC), :]
        blki = esi_s[pl.ds(j * _BLKC, _BLKC), :]
        gu = e0u_g[...] + jnp.dot(ums_g[...], wi_s[...],
                                  preferred_element_type=_f32)
        gi = e0i3[0:BS, :] + jnp.dot(vms_g[...], wu_s[...],
                                     preferred_element_type=_f32)
        dn_nn = (((1,), (1,)), ((), ()))
        lg_u = lax.dot_general(gu, blku, dn_nn,
                               preferred_element_type=_f32) / TEMP
        lg_i = lax.dot_general(gi, blki, dn_nn,
                               preferred_element_type=_f32) / TEMP
        seu_p = jnp.sum(jnp.exp(lg_u), axis=1, keepdims=True)
        sei_p = jnp.sum(jnp.exp(lg_i), axis=1, keepdims=True)

        @pl.when(i == _GRIDC)
        def _():
            se_u[...] = jnp.zeros_like(se_u)
            se_i[...] = jnp.zeros_like(se_i)

        se_u[...] += seu_p
        se_i[...] += sei_p

        @pl.when(i == 2 * _GRIDC - 1)
        def _():
            eusu = e0u_g[...] + a_g[...] * (ru1_g[...] + ru2_g[...])
            ei3 = e0i3[...] + b3[...] * (ri13[...] + ri23[...])
            eisi = ei3[0:BS, :]
            eisp = ei3[BS:2 * BS, :]
            eisn = ei3[2 * BS:3 * BS, :]
            pad = float(NT - N)
            neg = jnp.mean(jnp.log(se_u[...] - pad + 1e-08))
            neg += jnp.mean(jnp.log(se_i[...] - pad + 1e-08))
            pos = jnp.mean(jnp.clip(jnp.sum(gu * eusu, axis=1) / TEMP,
                                    -5.0, 5.0))
            pos += jnp.mean(jnp.clip(jnp.sum(gi * eisi, axis=1) / TEMP,
                                     -5.0, 5.0))
            loss_s = neg - pos
            ps = jnp.sum(eusu * eisp, axis=1)
            ns = jnp.sum(eusu * eisn, axis=1)
            sig = 1.0 / (1.0 + jnp.exp(-(ps - ns)))
            loss_r = -jnp.mean(jnp.log(sig + 1e-08))
            lossr_o[...] = jnp.reshape(loss_r, (1, 1))
            losss_o[...] = jnp.reshape(LAM1 * loss_s, (1, 1))
            loss_o[...] = jnp.reshape(loss_r + LAM1 * loss_s, (1, 1)) \
                + LAM2 * reg_s[...]


def _tc_c(e0u_p, r_u1, r_u2, a_v, utT_p, e0i_p, r_i1, r_i2, b_v, vtT_p,
          e0u_g, a_g, ru1_g, ru2_g, ums_g, vms_g, e0i3, b3, ri13, ri23):
    blk = lambda i: (jnp.where(i < _GRIDC, i, 0), 0)
    vec = pl.BlockSpec((_BLKC, 1), blk)
    mat = pl.BlockSpec((_BLKC, D), blk)
    fac = pl.BlockSpec((QP, _BLKC), lambda i: (0, jnp.where(i < _GRIDC, i, 0)))
    cst = lambda shape: pl.BlockSpec(shape, lambda i: (0, 0))
    return pl.pallas_call(
        _c_body,
        grid=(2 * _GRIDC,),
        in_specs=[mat, mat, mat, vec, fac, mat, mat, mat, vec, fac,
                  cst((BS, D)), cst((BS, 1)), cst((BS, D)), cst((BS, D)),
                  cst((BS, QP)), cst((BS, QP)),
                  cst((3 * BS, D)), cst((3 * BS, 1)),
                  cst((3 * BS, D)), cst((3 * BS, D))],
        out_specs=[cst((1, 1)), cst((1, 1)), cst((1, 1))],
        out_shape=[jax.ShapeDtypeStruct((1, 1), _f32)] * 3,
        scratch_shapes=[pltpu.VMEM((NT, D), _f32),
                        pltpu.VMEM((NT, D), _f32),
                        pltpu.VMEM((QP, D), _f32),
                        pltpu.VMEM((QP, D), _f32),
                        pltpu.VMEM((1, 1), _f32),
                        pltpu.VMEM((BS, 1), _f32),
                        pltpu.VMEM((BS, 1), _f32)],
    )(e0u_p, r_u1, r_u2, a_v, utT_p, e0i_p, r_i1, r_i2, b_v, vtT_p,
      e0u_g, a_g, ru1_g, ru2_g, ums_g, vms_g, e0i3, b3, ri13, ri23)


def kernel(uids, iids, pos, neg, edge_u, edge_i, edge_vals,
           E_u_0, E_i_0, u_mul_s, v_mul_s, ut, vt):
    del edge_vals
    padi = jnp.full((NEP - NE,), DUMMY, jnp.int32)
    eu_p = jnp.concatenate([edge_u.astype(jnp.int32), padi]) \
        .reshape(NEP // CH, CH)
    ei_p = jnp.concatenate([edge_i.astype(jnp.int32), padi]) \
        .reshape(NEP // CH, CH)
    e0u_p = jnp.pad(E_u_0, ((0, NT - N), (0, 0)))
    e0i_p = jnp.pad(E_i_0, ((0, NT - N), (0, 0)))

    deg_u, deg_i = _build_sc_degrees()(eu_p, ei_p)
    t_u1, t_i1, a_v, b_v, inv_u, inv_i = _tc_prep(deg_u, deg_i, e0u_p, e0i_p)
    sc_spmm = _build_sc_spmm()
    r_u1, r_i1 = sc_spmm(t_u1, t_i1, eu_p, ei_p)
    t_u2, t_i2 = _tc_scale2(r_u1, r_i1, inv_u, inv_i)
    r_u2, r_i2 = sc_spmm(t_u2, t_i2, eu_p, ei_p)

    utT_p = jnp.pad(ut, ((0, QP - QR), (0, NT - N)))
    vtT_p = jnp.pad(vt, ((0, QP - QR), (0, NT - N)))
    idx3 = jnp.concatenate([iids, pos, neg])
    loss, loss_r, loss_s = _tc_c(
        e0u_p, r_u1, r_u2, a_v, utT_p, e0i_p, r_i1, r_i2, b_v, vtT_p,
        E_u_0[uids], a_v[uids], r_u1[uids], r_u2[uids],
        jnp.pad(u_mul_s[uids], ((0, 0), (0, QP - QR))),
        jnp.pad(v_mul_s[iids], ((0, 0), (0, QP - QR))),
        E_i_0[idx3], b_v[idx3], r_i1[idx3], r_i2[idx3])
    return (loss[0, 0], loss_r[0, 0], loss_s[0, 0])
